# baseline (speedup 1.0000x reference)
import functools

import jax
import jax.numpy as jnp
from jax import lax
from jax.experimental import pallas as pl
from jax.experimental.pallas import tpu as pltpu
from jax.experimental.pallas import tpu_sc as plsc

B = 64
NB = 4
V = 100000
EOS = 2
REP = 1.2
BN = B * NB

NC, NS, LANES = 2, 16, 16
NW = NC * NS
RPT = BN // NW
BC = 4096
NFULL = 24
SLABS = 1664
VTAIL = V - NFULL * BC - SLABS
TOKP = 64
TOKVECS = TOKP // LANES
LCAP = 48
K = 2 * NB
NEG = -3.0e38
BIGI = 2**30


def _iota16(off):
    return lax.iota(jnp.int32, 16) + off


def _store1(ref, pos, val):
    lane0 = lax.iota(jnp.int32, LANES) == 0
    plsc.store_scatter(ref, [jnp.broadcast_to(pos, (LANES,))],
                       jnp.broadcast_to(val, (LANES,)), mask=lane0)


def _load1(ref, pos):
    g = plsc.load_gather(ref, [jnp.broadcast_to(pos, (LANES,))])
    return jnp.max(g)


def _sc_stage(scores, scores_tail, tokens_p):
    mesh = plsc.VectorSubcoreMesh(core_axis_name="c", subcore_axis_name="s")

    @functools.partial(
        pl.kernel,
        out_type=(
            jax.ShapeDtypeStruct((BN * K,), jnp.float32),
            jax.ShapeDtypeStruct((BN * K,), jnp.int32),
            jax.ShapeDtypeStruct((BN,), jnp.float32),
            jax.ShapeDtypeStruct((BN,), jnp.float32),
        ),
        mesh=mesh,
        compiler_params=pltpu.CompilerParams(needs_layout_passes=False),
        scratch_types=[
            pltpu.VMEM((RPT, BC), jnp.float32),
            pltpu.VMEM((RPT, BC), jnp.float32),
            pltpu.VMEM((RPT, SLABS), jnp.float32),
            pltpu.VMEM((RPT, VTAIL), jnp.float32),
            pltpu.VMEM((RPT, TOKP), jnp.int32),
            pltpu.VMEM((RPT * LANES * LCAP,), jnp.float32),
            pltpu.VMEM((RPT * LANES * LCAP,), jnp.int32),
            pltpu.VMEM((LANES,), jnp.float32),
            pltpu.VMEM((RPT * LANES,), jnp.float32),
            pltpu.VMEM((RPT * LANES,), jnp.float32),
            pltpu.VMEM((RPT * LANES,), jnp.float32),
            pltpu.VMEM((RPT * LANES,), jnp.float32),
            pltpu.VMEM((RPT * LANES,), jnp.int32),
            pltpu.VMEM((RPT * K,), jnp.float32),
            pltpu.VMEM((RPT * K,), jnp.int32),
            pltpu.VMEM((RPT,), jnp.float32),
            pltpu.VMEM((RPT,), jnp.float32),
            pltpu.SemaphoreType.DMA,
            pltpu.SemaphoreType.DMA,
        ],
    )
    def sc_kernel(scores_hbm, tail_hbm, tok_hbm, ov_hbm, oc_hbm, om_hbm,
                  os_hbm, bufA, bufB, bufS, bufU, tokbuf, candv, candc, srt,
                  strun, slmax, sm0, ssum, scnt, t8v, t8c, m8, s8,
                  semA, semB):
        wid = lax.axis_index("s") * NC + lax.axis_index("c")
        r0 = wid * RPT
        lane = lax.iota(jnp.int32, LANES)

        pltpu.sync_copy(tok_hbm.at[pl.ds(r0, RPT), :], tokbuf)

        def cinit(i, _):
            candv[pl.ds(i * LANES, LANES)] = jnp.full((LANES,), NEG, jnp.float32)
            return 0
        lax.fori_loop(0, RPT * LCAP, cinit, 0)

        def start(c, dst, sem, w):
            pltpu.async_copy(
                scores_hbm.at[pl.ds(r0, RPT), pl.ds(c * BC, w)], dst, sem)

        def wait(dst, sem, w):
            pltpu.make_async_copy(
                scores_hbm.at[pl.ds(0, RPT), pl.ds(0, w)], dst, sem).wait()

        def penalty(sbuf, r, c0, w):
            for t in range(TOKVECS):
                tok = tokbuf[r, pl.ds(t * LANES, LANES)]
                loc = tok - c0
                msk = (loc >= 0) & (loc < w)
                locc = jnp.where(msk, loc, 0)
                rsp = jnp.broadcast_to(r, (LANES,))
                g = plsc.load_gather(sbuf, [rsp, locc], mask=msk)
                pen = jnp.where(g < 0.0, g * REP, g * (1.0 / REP))
                plsc.store_scatter(sbuf, [rsp, locc], pen, mask=msk)

        def splat8th(lmax):
            srt[...] = plsc.sort_key_val(lmax, lmax, descending=True)[0]
            return jnp.broadcast_to(_load1(srt, K - 1), (LANES,))

        def process(sbuf, c0, w, do_penalty=True):
            nv = w // LANES

            def rbody(r, _):
                if do_penalty:
                    penalty(sbuf, r, c0, w)
                rb = r * LANES
                ringb = lane * LCAP + r * (LANES * LCAP)
                t_run = strun[pl.ds(rb, LANES)]
                m0 = sm0[pl.ds(rb, LANES)]

                def sweep(i, car):
                    lmax, cnt = car
                    x = sbuf[r, pl.ds(i * LANES, LANES)]
                    lmax = jnp.maximum(lmax, x)
                    msk = x >= t_run
                    idx = ringb + jnp.minimum(cnt, LCAP - 1)
                    plsc.store_scatter(candv, [idx], x, mask=msk)
                    plsc.store_scatter(candc, [idx], lane + (c0 + i * LANES),
                                       mask=msk)
                    return lmax, cnt + msk.astype(jnp.int32)

                lmax, cnt = lax.fori_loop(
                    0, nv, sweep,
                    (slmax[pl.ds(rb, LANES)], scnt[pl.ds(rb, LANES)]),
                    unroll=8)

                def esweep(i, sacc):
                    x = sbuf[r, pl.ds(i * LANES, LANES)]
                    return sacc + jnp.exp(x - m0)
                sacc = lax.fori_loop(0, nv, esweep, ssum[pl.ds(rb, LANES)],
                                     unroll=8)
                slmax[pl.ds(rb, LANES)] = lmax
                ssum[pl.ds(rb, LANES)] = sacc
                scnt[pl.ds(rb, LANES)] = cnt
                strun[pl.ds(rb, LANES)] = splat8th(lmax)
                return 0

            lax.fori_loop(0, RPT, rbody, 0)

        def process0(sbuf, w):
            nv = w // LANES

            def rbody(r, _):
                penalty(sbuf, r, 0, w)
                rb = r * LANES

                def bmax(i, mv):
                    return jnp.maximum(mv, sbuf[r, pl.ds(i * LANES, LANES)])
                lmax = lax.fori_loop(0, nv, bmax,
                                     jnp.full((LANES,), NEG, jnp.float32),
                                     unroll=8)
                sm0[pl.ds(rb, LANES)] = jnp.broadcast_to(jnp.max(lmax),
                                                         (LANES,))
                strun[pl.ds(rb, LANES)] = splat8th(lmax)
                slmax[pl.ds(rb, LANES)] = jnp.full((LANES,), NEG, jnp.float32)
                ssum[pl.ds(rb, LANES)] = jnp.zeros((LANES,), jnp.float32)
                scnt[pl.ds(rb, LANES)] = jnp.zeros((LANES,), jnp.int32)
                return 0

            lax.fori_loop(0, RPT, rbody, 0)
            process(sbuf, 0, w, do_penalty=False)

        start(0, bufA, semA, BC)
        start(1, bufB, semB, BC)
        wait(bufA, semA, BC)
        process0(bufA, BC)
        start(2, bufA, semA, BC)

        def group(g, _):
            c = 2 * g + 1
            wait(bufB, semB, BC)
            process(bufB, c * BC, BC)
            start(c + 2, bufB, semB, BC)
            wait(bufA, semA, BC)
            process(bufA, (c + 1) * BC, BC)
            lax.cond(c + 3 <= NFULL - 1,
                     lambda: start(c + 3, bufA, semA, BC), lambda: None)
            return 0

        lax.fori_loop(0, (NFULL - 2) // 2, group, 0)
        wait(bufB, semB, BC)
        pltpu.async_copy(
            scores_hbm.at[pl.ds(r0, RPT), pl.ds(NFULL * BC, SLABS)],
            bufS, semA)
        process(bufB, (NFULL - 1) * BC, BC)
        pltpu.make_async_copy(
            scores_hbm.at[pl.ds(0, RPT), pl.ds(0, SLABS)], bufS, semA).wait()
        pltpu.async_copy(tail_hbm.at[pl.ds(r0, RPT), :], bufU, semB)
        process(bufS, NFULL * BC, SLABS)
        pltpu.make_async_copy(
            tail_hbm.at[pl.ds(0, RPT), :], bufU, semB).wait()
        process(bufU, NFULL * BC + SLABS, VTAIL)

        def drow(r, _):
            rbase = r * (LANES * LCAP)
            rb = r * LANES

            def dstep(j, _2):
                def body(i, car):
                    bv, bc, bp = car
                    x = candv[pl.ds(rbase + i * LANES, LANES)]
                    c = candc[pl.ds(rbase + i * LANES, LANES)]
                    upd = (x > bv) | ((x == bv) & (c < bc))
                    return (jnp.where(upd, x, bv), jnp.where(upd, c, bc),
                            jnp.where(upd, _iota16(i * LANES), bp))
                bv, bc, bp = lax.fori_loop(
                    0, LCAP, body,
                    (jnp.full((LANES,), NEG, jnp.float32),
                     jnp.full((LANES,), BIGI, jnp.int32),
                     jnp.full((LANES,), BIGI, jnp.int32)), unroll=4)
                m = jnp.max(bv)
                hit = bv >= m
                col = jnp.min(jnp.where(hit, bc, BIGI))
                pos = jnp.min(jnp.where(hit & (bc == col), bp, BIGI))
                _store1(candv, rbase + pos, jnp.float32(NEG))
                _store1(t8v, r * K + j, m)
                _store1(t8c, r * K + j, col)
                return 0
            lax.fori_loop(0, K, dstep, 0)

            m_row = jnp.max(slmax[pl.ds(rb, LANES)])
            _store1(m8, r, m_row)
            scale = jnp.exp(sm0[pl.ds(rb, LANES)]
                            - jnp.broadcast_to(m_row, (LANES,)))
            _store1(s8, r, jnp.sum(ssum[pl.ds(rb, LANES)] * scale))
            return 0
        lax.fori_loop(0, RPT, drow, 0)

        pltpu.sync_copy(t8v, ov_hbm.at[pl.ds(r0 * K, RPT * K)])
        pltpu.sync_copy(t8c, oc_hbm.at[pl.ds(r0 * K, RPT * K)])
        pltpu.sync_copy(m8, om_hbm.at[pl.ds(r0, RPT)])
        pltpu.sync_copy(s8, os_hbm.at[pl.ds(r0, RPT)])

    return sc_kernel(scores, scores_tail, tokens_p)


def _tc_merge_body(v_ref, c_ref, m_ref, s_ref, b_ref,
                   ks_ref, kt_ref, kb_ref, ri_ref):
    x = v_ref[:]
    cols = c_ref[:]
    logp = x - m_ref[:] - jnp.log(s_ref[:]) + b_ref[:]
    pos = lax.broadcasted_iota(jnp.int32, (B, NB * K), 1)

    remaining = logp
    nb_seen = jnp.zeros((B, 1), jnp.int32)
    ks = [jnp.zeros((B, 1), jnp.float32) for _ in range(NB)]
    kt = [jnp.zeros((B, 1), jnp.int32) for _ in range(NB)]
    kb = [jnp.zeros((B, 1), jnp.int32) for _ in range(NB)]
    for _ in range(K):
        mx = jnp.max(remaining, axis=1, keepdims=True)
        p = jnp.min(jnp.where(remaining >= mx, pos, BIGI), axis=1, keepdims=True)
        hit = pos == p
        tok = jnp.sum(jnp.where(hit, cols, 0), axis=1, keepdims=True)
        beam = p // K
        remaining = jnp.where(hit, NEG, remaining)
        not_eos = (tok != EOS).astype(jnp.int32)
        rank = nb_seen + not_eos
        for q in range(NB):
            keep = (not_eos == 1) & (rank == q + 1)
            ks[q] = jnp.where(keep, mx, ks[q])
            kt[q] = jnp.where(keep, tok, kt[q])
            kb[q] = jnp.where(keep, beam, kb[q])
        nb_seen = nb_seen + not_eos

    ks_ref[:] = jnp.concatenate(ks, axis=1)
    kt_ref[:] = jnp.concatenate(kt, axis=1)
    kbm = jnp.concatenate(kb, axis=1)
    kb_ref[:] = kbm
    ri_ref[:] = kbm + jnp.arange(B, dtype=jnp.int32)[:, None] * NB


def _tc_merge(vals, cols, m, s, beam_scores):
    v32 = vals.reshape(B, NB * K)
    c32 = cols.reshape(B, NB * K)
    m32 = jnp.repeat(m.reshape(B, NB), K, axis=1)
    s32 = jnp.repeat(s.reshape(B, NB), K, axis=1)
    b32 = jnp.repeat(beam_scores.reshape(B, NB), K, axis=1)
    ks, kt, kb, ri = pl.pallas_call(
        _tc_merge_body,
        out_shape=(
            jax.ShapeDtypeStruct((B, NB), jnp.float32),
            jax.ShapeDtypeStruct((B, NB), jnp.int32),
            jax.ShapeDtypeStruct((B, NB), jnp.int32),
            jax.ShapeDtypeStruct((B, NB), jnp.int32),
        ),
    )(v32, c32, m32, s32, b32)
    return ks, kt, kb, ri.reshape(-1)


def kernel(scores, beam_scores, token_ids):
    tokens_p = jnp.pad(token_ids, ((0, 0), (0, TOKP - token_ids.shape[1])),
                       constant_values=BIGI)
    scores_tail = lax.slice(scores, (0, NFULL * BC + SLABS), (BN, V))
    vals, cols, m, s = _sc_stage(scores, scores_tail, tokens_p)
    return _tc_merge(vals, cols, m, s, beam_scores)


# speedup vs baseline: 1.1986x; 1.1986x over previous
import functools

import jax
import jax.numpy as jnp
from jax import lax
from jax.experimental import pallas as pl
from jax.experimental.pallas import tpu as pltpu
from jax.experimental.pallas import tpu_sc as plsc

B = 64
NB = 4
V = 100000
EOS = 2
REP = 1.2
BN = B * NB

NC, NS, LANES = 2, 16, 16
NW = NC * NS
ROWS_PER_TILE = BN // NW
CHUNK = 2000
NCHUNK = V // CHUNK
CVECS = CHUNK // LANES
TOKP = 64
TOKVECS = TOKP // LANES
CMAXP = 64
CAND_CAP = 256
CANDVECS = CAND_CAP // LANES
DEPTH = 5
K = 2 * NB
NEG = -3.0e38
BIGI = 2**30


def _iota16(off):
    return lax.iota(jnp.int32, 16) + off


def _store1(ref, pos, val):
    lane0 = lax.iota(jnp.int32, LANES) == 0
    plsc.store_scatter(ref, [jnp.broadcast_to(pos, (LANES,))],
                       jnp.broadcast_to(val, (LANES,)), mask=lane0)


def _load1(ref, pos):
    g = plsc.load_gather(ref, [jnp.broadcast_to(pos, (LANES,))])
    return jnp.max(g)


def _argmax_sweep(ref, nvecs):
    def body(i, carry):
        mv, av = carry
        x = ref[pl.ds(i * LANES, LANES)]
        idx = _iota16(i * LANES)
        upd = x > mv
        return jnp.maximum(mv, x), jnp.where(upd, idx, av)
    mv, av = lax.fori_loop(
        0, nvecs, body,
        (jnp.full((LANES,), NEG, jnp.float32), jnp.full((LANES,), BIGI, jnp.int32)),
        unroll=4)
    m = jnp.max(mv)
    pos = jnp.min(jnp.where(mv >= m, av, BIGI))
    return m, pos


def _sc_stage(scores, tokens_p):
    mesh = plsc.VectorSubcoreMesh(core_axis_name="c", subcore_axis_name="s")

    @functools.partial(
        pl.kernel,
        out_type=(
            jax.ShapeDtypeStruct((BN * K,), jnp.float32),
            jax.ShapeDtypeStruct((BN * K,), jnp.int32),
            jax.ShapeDtypeStruct((BN,), jnp.float32),
            jax.ShapeDtypeStruct((BN,), jnp.float32),
        ),
        mesh=mesh,
        compiler_params=pltpu.CompilerParams(needs_layout_passes=False),
        scratch_types=[
            pltpu.VMEM((V,), jnp.float32),
            pltpu.VMEM((TOKP,), jnp.int32),
            pltpu.VMEM((CMAXP,), jnp.float32),
            pltpu.VMEM((CMAXP,), jnp.float32),
            pltpu.VMEM((CAND_CAP,), jnp.float32),
            pltpu.VMEM((CAND_CAP,), jnp.int32),
            pltpu.VMEM((ROWS_PER_TILE * K,), jnp.float32),
            pltpu.VMEM((ROWS_PER_TILE * K,), jnp.int32),
            pltpu.VMEM((ROWS_PER_TILE,), jnp.float32),
            pltpu.VMEM((ROWS_PER_TILE,), jnp.float32),
        ] + [pltpu.SemaphoreType.DMA] * DEPTH,
    )
    def sc_kernel(scores_hbm, tok_hbm, ov_hbm, oc_hbm, om_hbm, os_hbm,
                  buf, tokbuf, cmax, cmax2, candv, candc, t8v, t8c, m8, s8,
                  *sems):
        wid = lax.axis_index("s") * NC + lax.axis_index("c")
        base = wid * ROWS_PER_TILE

        def do_row(k_row, _):
            row = base + k_row
            pltpu.sync_copy(tok_hbm.at[pl.ds(row * TOKP, TOKP)], tokbuf)
            for i in range(CMAXP // LANES):
                cmax[pl.ds(i * LANES, LANES)] = jnp.full((LANES,), NEG, jnp.float32)
            for i in range(CANDVECS):
                candv[pl.ds(i * LANES, LANES)] = jnp.full((LANES,), NEG, jnp.float32)

            def start(c, sem):
                pltpu.async_copy(
                    scores_hbm.at[pl.ds(row * V + c * CHUNK, CHUNK)],
                    buf.at[pl.ds(c * CHUNK, CHUNK)], sem)

            def wait(sem):
                pltpu.make_async_copy(scores_hbm.at[pl.ds(0, CHUNK)],
                                      buf.at[pl.ds(0, CHUNK)], sem).wait()

            for d in range(DEPTH):
                start(d, sems[d])

            def process(c, carry):
                m_run, s_run = carry
                c0 = c * CHUNK
                for t in range(TOKVECS):
                    tok = tokbuf[pl.ds(t * LANES, LANES)]
                    msk = (tok >= c0) & (tok < c0 + CHUNK)
                    tokc = jnp.where(msk, tok, 0)
                    g = plsc.load_gather(buf, [tokc], mask=msk)
                    pen = jnp.where(g < 0.0, g * REP, g * (1.0 / REP))
                    plsc.store_scatter(buf, [tokc], pen, mask=msk)

                def bmax(i, mv):
                    return jnp.maximum(mv, buf[pl.ds(c0 + i * LANES, LANES)])
                mv = lax.fori_loop(0, CVECS, bmax,
                                   jnp.full((LANES,), NEG, jnp.float32),
                                   unroll=25)
                m_c = jnp.max(mv)
                _store1(cmax, c, m_c)
                m_new = jnp.maximum(m_run, m_c)
                s_run = s_run * jnp.exp(jnp.broadcast_to(m_run - m_new, (LANES,)))

                def bsum(i, acc):
                    x = buf[pl.ds(c0 + i * LANES, LANES)]
                    return acc + jnp.exp(x - m_new)
                s_run = lax.fori_loop(0, CVECS, bsum, s_run, unroll=25)
                return m_new, s_run

            def group_a(g, carry):
                for d in range(DEPTH):
                    c = g * DEPTH + d
                    wait(sems[d])
                    nxt = c + DEPTH
                    lax.cond(nxt < NCHUNK,
                             lambda: start(nxt, sems[d]), lambda: None)
                    carry = process(c, carry)
                return carry

            m_row, s_vec = lax.fori_loop(
                0, NCHUNK // DEPTH, group_a,
                (jnp.float32(NEG), jnp.zeros((LANES,), jnp.float32)))

            for i in range(CMAXP // LANES):
                cmax2[pl.ds(i * LANES, LANES)] = cmax[pl.ds(i * LANES, LANES)]

            def tstep(j, t):
                m, pos = _argmax_sweep(cmax, CMAXP // LANES)
                _store1(cmax, pos, jnp.float32(NEG))
                return m
            t_thr = lax.fori_loop(0, K, tstep, jnp.float32(NEG))

            def chunk_c(c, cnt):
                c0 = c * CHUNK

                def collect(cnt):
                    def bcol(i, cnt):
                        x = buf[pl.ds(c0 + i * LANES, LANES)]
                        msk = x >= t_thr
                        cnt = jnp.minimum(cnt, CAND_CAP - LANES)
                        plsc.store_compressed(candv.at[pl.ds(cnt, LANES)], x,
                                              mask=msk)
                        plsc.store_compressed(candc.at[pl.ds(cnt, LANES)],
                                              _iota16(c0 + i * LANES), mask=msk)
                        return cnt + jnp.sum(msk.astype(jnp.int32))
                    return lax.fori_loop(0, CVECS, bcol, cnt)

                return lax.cond(_load1(cmax2, c) >= t_thr, collect,
                                lambda n: n, cnt)

            lax.fori_loop(0, NCHUNK, chunk_c, jnp.int32(0))

            def dstep(j, _):
                m, pos = _argmax_sweep(candv, CANDVECS)
                col = _load1(candc, pos)
                _store1(candv, pos, jnp.float32(NEG))
                _store1(t8v, k_row * K + j, m)
                _store1(t8c, k_row * K + j, col)
                return 0
            lax.fori_loop(0, K, dstep, 0)

            _store1(m8, k_row, m_row)
            _store1(s8, k_row, jnp.sum(s_vec))
            return 0

        lax.fori_loop(0, ROWS_PER_TILE, do_row, 0)
        pltpu.sync_copy(t8v, ov_hbm.at[pl.ds(base * K, ROWS_PER_TILE * K)])
        pltpu.sync_copy(t8c, oc_hbm.at[pl.ds(base * K, ROWS_PER_TILE * K)])
        pltpu.sync_copy(m8, om_hbm.at[pl.ds(base, ROWS_PER_TILE)])
        pltpu.sync_copy(s8, os_hbm.at[pl.ds(base, ROWS_PER_TILE)])

    return sc_kernel(scores.reshape(-1), tokens_p.reshape(-1))


def _tc_merge_body(v_ref, c_ref, m_ref, s_ref, b_ref,
                   ks_ref, kt_ref, kb_ref, ri_ref):
    x = v_ref[:]
    cols = c_ref[:]
    logp = x - m_ref[:] - jnp.log(s_ref[:]) + b_ref[:]
    pos = lax.broadcasted_iota(jnp.int32, (B, NB * K), 1)

    remaining = logp
    nb_seen = jnp.zeros((B, 1), jnp.int32)
    ks = [jnp.zeros((B, 1), jnp.float32) for _ in range(NB)]
    kt = [jnp.zeros((B, 1), jnp.int32) for _ in range(NB)]
    kb = [jnp.zeros((B, 1), jnp.int32) for _ in range(NB)]
    for _ in range(K):
        mx = jnp.max(remaining, axis=1, keepdims=True)
        p = jnp.min(jnp.where(remaining >= mx, pos, BIGI), axis=1, keepdims=True)
        hit = pos == p
        tok = jnp.sum(jnp.where(hit, cols, 0), axis=1, keepdims=True)
        beam = p // K
        remaining = jnp.where(hit, NEG, remaining)
        not_eos = (tok != EOS).astype(jnp.int32)
        rank = nb_seen + not_eos
        for q in range(NB):
            keep = (not_eos == 1) & (rank == q + 1)
            ks[q] = jnp.where(keep, mx, ks[q])
            kt[q] = jnp.where(keep, tok, kt[q])
            kb[q] = jnp.where(keep, beam, kb[q])
        nb_seen = nb_seen + not_eos

    ks_ref[:] = jnp.concatenate(ks, axis=1)
    kt_ref[:] = jnp.concatenate(kt, axis=1)
    kbm = jnp.concatenate(kb, axis=1)
    kb_ref[:] = kbm
    ri_ref[:] = kbm + jnp.arange(B, dtype=jnp.int32)[:, None] * NB


def _tc_merge(vals, cols, m, s, beam_scores):
    v32 = vals.reshape(B, NB * K)
    c32 = cols.reshape(B, NB * K)
    m32 = jnp.repeat(m.reshape(B, NB), K, axis=1)
    s32 = jnp.repeat(s.reshape(B, NB), K, axis=1)
    b32 = jnp.repeat(beam_scores.reshape(B, NB), K, axis=1)
    ks, kt, kb, ri = pl.pallas_call(
        _tc_merge_body,
        out_shape=(
            jax.ShapeDtypeStruct((B, NB), jnp.float32),
            jax.ShapeDtypeStruct((B, NB), jnp.int32),
            jax.ShapeDtypeStruct((B, NB), jnp.int32),
            jax.ShapeDtypeStruct((B, NB), jnp.int32),
        ),
    )(v32, c32, m32, s32, b32)
    return ks, kt, kb, ri.reshape(-1)


def kernel(scores, beam_scores, token_ids):
    tokens_p = jnp.pad(token_ids, ((0, 0), (0, TOKP - token_ids.shape[1])),
                       constant_values=BIGI)
    vals, cols, m, s = _sc_stage(scores, tokens_p)
    return _tc_merge(vals, cols, m, s, beam_scores)
